# Initial kernel scaffold; baseline (speedup 1.0000x reference)
import functools

import jax
import jax.numpy as jnp
from jax import lax
from jax.experimental import pallas as pl
from jax.experimental.pallas import tpu as pltpu
from jax.experimental.pallas import tpu_sc as plsc

_N = 10000
_E = 160000
_NTILES = 32
_EPT = _E // _NTILES
_CH = 1000
_NPT = _N // 16
_BE = 2000
_BN = 2000


def _edge_mlp_block(ef, w1, b1, w2, b2, w3, b3, out):
    t = jnp.maximum(ef[...] @ w1[...] + b1[...], 0.0)
    t = jnp.maximum(t @ w2[...] + b2[...], 0.0)
    t = jnp.maximum(t @ w3[...] + b3[...], 0.0)
    out[:, 0:32] = t
    col = lax.broadcasted_iota(jnp.int32, (t.shape[0], 16), 1)
    out[:, 32:48] = jnp.where(col == 0, 1.0, 0.0).astype(jnp.float32)


def _node_block(nf, wn, bn_, out):
    out[...] = jnp.maximum(jnp.dot(nf[...], wn[...]) + bn_[...], 0.0)


def _layer_block(x, g0, g1, e0, e1, wx, we, wl, lb, out):
    g = g0[...] + g1[...]
    es = e0[...] + e1[...]
    acc = jnp.dot(g, wx[...]) + jnp.dot(es, we[...]) + jnp.dot(x[...], wl[...]) + lb[...]
    out[...] = jnp.maximum(acc, 0.0)


def _final_block(x, g0, g1, e0, e1, wx, we, wl, lb, wlast, blast, out):
    g = g0[...] + g1[...]
    es = e0[...] + e1[...]
    acc = jnp.dot(g, wx[...]) + jnp.dot(es, we[...]) + jnp.dot(x[...], wl[...]) + lb[...]
    x5 = jnp.maximum(acc, 0.0)
    out[...] = jnp.sum(x5 * wlast[...], axis=1, keepdims=True) + blast[...]


def _full(shape):
    return pl.BlockSpec(shape, lambda i: (0, 0))


def _rows(shape):
    return pl.BlockSpec(shape, lambda i: (i, 0))


def _sc_mesh():
    return plsc.VectorSubcoreMesh(core_axis_name="c", subcore_axis_name="s")


def _zero_rows(buf, nrows, ncols):
    def body(i, carry):
        for j in range(ncols // 16):
            buf[i, pl.ds(j * 16, 16)] = jnp.zeros((16,), jnp.float32)
        return carry
    lax.fori_loop(0, nrows, body, 0)


@functools.partial(
    pl.kernel,
    out_type=(jax.ShapeDtypeStruct((2, _N, 48), jnp.float32),
              jax.ShapeDtypeStruct((2, _N, 16), jnp.float32)),
    mesh=_sc_mesh(),
    scratch_types=[
        pltpu.VMEM((_CH,), jnp.int32),
        pltpu.VMEM((_CH,), jnp.int32),
        pltpu.VMEM((_CH, 48), jnp.float32),
        pltpu.VMEM((_CH, 16), jnp.float32),
        pltpu.VMEM_SHARED((_N, 48), jnp.float32),
        pltpu.VMEM_SHARED((_N, 16), jnp.float32),
        pltpu.SemaphoreType.DMA,
    ],
)
def _sc_layer1(e_hbm, h_hbm, src_hbm, dst_hbm, eout_hbm, gout_hbm,
               sidx, didx, erows, hrows, eacc, gacc, sem):
    cid = lax.axis_index("c")
    sid = lax.axis_index("s")
    wid = sid * 2 + cid
    _zero_rows(erows, _NPT, 48)
    _zero_rows(hrows, _NPT, 16)
    pltpu.sync_copy(erows.at[pl.ds(0, _NPT)], eacc.at[pl.ds(sid * _NPT, _NPT)])
    pltpu.sync_copy(hrows.at[pl.ds(0, _NPT)], gacc.at[pl.ds(sid * _NPT, _NPT)])
    plsc.subcore_barrier()

    def chunk(k, carry):
        base = pl.multiple_of(wid * _EPT + k * _CH, 8)
        pltpu.sync_copy(src_hbm.at[pl.ds(base, _CH)], sidx)
        pltpu.sync_copy(dst_hbm.at[pl.ds(base, _CH)], didx)
        pltpu.sync_copy(e_hbm.at[pl.ds(base, _CH)], erows)
        pltpu.async_copy(h_hbm.at[sidx], hrows, sem).wait()
        pltpu.sync_copy(erows, eacc.at[didx], add=True)
        pltpu.sync_copy(hrows, gacc.at[didx], add=True)
        return carry
    lax.fori_loop(0, _EPT // _CH, chunk, 0)

    plsc.subcore_barrier()
    pltpu.sync_copy(eacc.at[pl.ds(sid * _NPT, _NPT)],
                    eout_hbm.at[cid, pl.ds(sid * _NPT, _NPT)])
    pltpu.sync_copy(gacc.at[pl.ds(sid * _NPT, _NPT)],
                    gout_hbm.at[cid, pl.ds(sid * _NPT, _NPT)])


@functools.partial(
    pl.kernel,
    out_type=jax.ShapeDtypeStruct((2, _N, 32), jnp.float32),
    mesh=_sc_mesh(),
    scratch_types=[
        pltpu.VMEM((_CH,), jnp.int32),
        pltpu.VMEM((_CH,), jnp.int32),
        pltpu.VMEM((_CH, 32), jnp.float32),
        pltpu.VMEM_SHARED((_N, 32), jnp.float32),
        pltpu.SemaphoreType.DMA,
    ],
)
def _sc_seg_gather(x_hbm, src_hbm, dst_hbm, out_hbm, sidx, didx, rows, acc, sem):
    cid = lax.axis_index("c")
    sid = lax.axis_index("s")
    wid = sid * 2 + cid
    _zero_rows(rows, _NPT, 32)
    pltpu.sync_copy(rows.at[pl.ds(0, _NPT)], acc.at[pl.ds(sid * _NPT, _NPT)])
    plsc.subcore_barrier()

    def chunk(k, carry):
        base = pl.multiple_of(wid * _EPT + k * _CH, 8)
        pltpu.sync_copy(src_hbm.at[pl.ds(base, _CH)], sidx)
        pltpu.sync_copy(dst_hbm.at[pl.ds(base, _CH)], didx)
        pltpu.async_copy(x_hbm.at[sidx], rows, sem).wait()
        pltpu.sync_copy(rows, acc.at[didx], add=True)
        return carry
    lax.fori_loop(0, _EPT // _CH, chunk, 0)

    plsc.subcore_barrier()
    pltpu.sync_copy(acc.at[pl.ds(sid * _NPT, _NPT)],
                    out_hbm.at[cid, pl.ds(sid * _NPT, _NPT)])


def kernel(node_feature, edge_feature, edge_index,
           Wn, bn, We1, be1, We2, be2, We3, be3,
           netW1, netb1, linW1, linb1,
           netW2, netb2, linW2, linb2,
           netW3, netb3, linW3, linb3,
           netW4, netb4, linW4, linb4,
           netW5, netb5, linW5, linb5,
           Wlast, blast):
    src = edge_index[0]
    dst = edge_index[1]

    e_aug = pl.pallas_call(
        _edge_mlp_block,
        grid=(_E // _BE,),
        in_specs=[_rows((_BE, 301)),
                  _full((301, 128)), _full((1, 128)),
                  _full((128, 64)), _full((1, 64)),
                  _full((64, 32)), _full((1, 32))],
        out_specs=_rows((_BE, 48)),
        out_shape=jax.ShapeDtypeStruct((_E, 48), jnp.float32),
    )(edge_feature, We1.T, be1.reshape(1, -1),
      We2.T, be2.reshape(1, -1), We3.T, be3.reshape(1, -1))

    h = pl.pallas_call(
        _node_block,
        grid=(1,),
        in_specs=[_full((_N, 3)), _full((3, 16)), _full((1, 16))],
        out_specs=_full((_N, 16)),
        out_shape=jax.ShapeDtypeStruct((_N, 16), jnp.float32),
    )(node_feature, Wn.T, bn.reshape(1, -1))

    esum_p, g_p = _sc_layer1(e_aug, h, src, dst)
    e0, e1 = esum_p[0], esum_p[1]

    convs = [(netW1, netb1, linW1, linb1), (netW2, netb2, linW2, linb2),
             (netW3, netb3, linW3, linb3), (netW4, netb4, linW4, linb4),
             (netW5, netb5, linW5, linb5)]

    x = h
    out2d = None
    for li, (nW, nb, lW, lb) in enumerate(convs):
        f_in = lW.shape[1]
        wx = nW[:, :f_in].T
        we_aug = jnp.concatenate(
            [nW[:, f_in:], nb[:, None], jnp.zeros((32, 15), jnp.float32)],
            axis=1).T
        wl = lW.T
        lbr = lb.reshape(1, -1)
        if li > 0:
            g_p = _sc_seg_gather(x, src, dst)
        g0, g1 = g_p[0], g_p[1]
        common_specs = [_rows((_BN, f_in)), _rows((_BN, f_in)), _rows((_BN, f_in)),
                        _rows((_BN, 48)), _rows((_BN, 48)),
                        _full((f_in, 32)), _full((48, 32)),
                        _full((f_in, 32)), _full((1, 32))]
        if li < 4:
            x = pl.pallas_call(
                _layer_block,
                grid=(_N // _BN,),
                in_specs=common_specs,
                out_specs=_rows((_BN, 32)),
                out_shape=jax.ShapeDtypeStruct((_N, 32), jnp.float32),
            )(x, g0, g1, e0, e1, wx, we_aug, wl, lbr)
        else:
            out2d = pl.pallas_call(
                _final_block,
                grid=(_N // _BN,),
                in_specs=common_specs + [_full((1, 32)), _full((1, 1))],
                out_specs=_rows((_BN, 1)),
                out_shape=jax.ShapeDtypeStruct((_N, 1), jnp.float32),
            )(x, g0, g1, e0, e1, wx, we_aug, wl, lbr, Wlast, blast.reshape(1, 1))

    return out2d[:, 0]


# speedup vs baseline: 3.2159x; 3.2159x over previous
import functools

import jax
import jax.numpy as jnp
from jax import lax
from jax.experimental import pallas as pl
from jax.experimental.pallas import tpu as pltpu
from jax.experimental.pallas import tpu_sc as plsc

_N = 10000
_E = 160000
_W = 128
_CH = 128
_NPW = 632
_HC = 48
_BE = 2000
_BN = 2000


def _edge_mlp_block(ef, w1, b1, w2, b2, w3, b3, out):
    t = jnp.maximum(ef[...] @ w1[...] + b1[...], 0.0)
    t = jnp.maximum(t @ w2[...] + b2[...], 0.0)
    t = jnp.maximum(t @ w3[...] + b3[...], 0.0)
    n = t.shape[0]
    out[...] = jnp.concatenate(
        [t, jnp.ones((n, 1), jnp.float32), jnp.zeros((n, _W - 33), jnp.float32)],
        axis=1)


def _node_block(nf, wn, bn_, out):
    h = jnp.maximum(jnp.dot(nf[...], wn[...]) + bn_[...], 0.0)
    out[...] = jnp.concatenate(
        [h, jnp.zeros((h.shape[0], _W - 16), jnp.float32)], axis=1)


def _layer_block(x, e0, e1, g0, g1, we, wx, wl, lb, out):
    es = e0[0] + e1[0]
    g = g0[0] + g1[0]
    acc = jnp.dot(es, we[...]) + jnp.dot(g, wx[...]) \
        + jnp.dot(x[...], wl[...]) + lb[...]
    r = jnp.maximum(acc, 0.0)
    out[...] = jnp.concatenate(
        [r, jnp.zeros((r.shape[0], _W - 32), jnp.float32)], axis=1)


def _final_block(x, e0, e1, g0, g1, we, wx, wl, lb, wlast, blast, out):
    es = e0[0] + e1[0]
    g = g0[0] + g1[0]
    acc = jnp.dot(es, we[...]) + jnp.dot(g, wx[...]) \
        + jnp.dot(x[...], wl[...]) + lb[...]
    x5 = jnp.maximum(acc, 0.0)
    out[...] = jnp.sum(x5 * wlast[...], axis=1, keepdims=True) + blast[...]


def _full(shape):
    return pl.BlockSpec(shape, lambda i: (0,) * len(shape))


def _rows(shape):
    return pl.BlockSpec(shape, lambda i: (i,) + (0,) * (len(shape) - 1))


def _part(c, shape):
    return pl.BlockSpec((1,) + shape, lambda i, _c=c: (_c, i, 0))


def _zero_buf(buf):
    def body(i, carry):
        for j in range(_W // 16):
            buf[i, pl.ds(j * 16, 16)] = jnp.zeros((16,), jnp.float32)
        return carry
    lax.fori_loop(0, buf.shape[0], body, 0)


def _zero_shared_rows(buf, acc, row0):
    off = 0
    while off < _NPW:
        nr = min(_CH, _NPW - off)
        pltpu.sync_copy(buf.at[pl.ds(0, nr)], acc.at[pl.ds(row0 + off, nr)])
        off += nr


def _tile_chunks(wid):
    nblk = jnp.where(wid < 2, 40, 39)
    start = wid * 4992 + 128 * jnp.minimum(wid, 2)
    return nblk, start


@functools.cache
def _get_sc_layer1():
    return functools.partial(
        pl.kernel,
        out_type=jax.ShapeDtypeStruct((2, _N, _W), jnp.float32),
        mesh=plsc.VectorSubcoreMesh(core_axis_name="c", subcore_axis_name="s"),
        scratch_types=[
            pltpu.VMEM((_CH,), jnp.int32),
            pltpu.VMEM((_CH,), jnp.int32),
            pltpu.VMEM((_CH, _W), jnp.float32),
            pltpu.VMEM((_CH, _W), jnp.float32),
            pltpu.VMEM_SHARED((_N, _W), jnp.float32),
            pltpu.SemaphoreType.DMA,
        ],
    )(_sc_layer1_body)


def _sc_layer1_body(e_hbm, h_hbm, src_hbm, dst_hbm, out_hbm,
                    sidx, didx, erows, hrows, acc, sem):
    cid = lax.axis_index("c")
    sid = lax.axis_index("s")
    wid = sid * 2 + cid
    row0 = pl.multiple_of(jnp.minimum(sid * _NPW, _N - _NPW), 8)
    _zero_buf(erows)
    _zero_shared_rows(erows, acc, row0)
    plsc.subcore_barrier()

    nblk, start = _tile_chunks(wid)

    def chunk(k, carry):
        base = pl.multiple_of(start + k * _CH, 8)
        pltpu.sync_copy(src_hbm.at[pl.ds(base, _CH)], sidx)
        pltpu.sync_copy(dst_hbm.at[pl.ds(base, _CH)], didx)
        pltpu.sync_copy(e_hbm.at[pl.ds(base, _CH)], erows)
        pltpu.async_copy(h_hbm.at[sidx], hrows, sem).wait()

        def merge(i, c):
            erows[i, pl.ds(_HC, 16)] = hrows[i, pl.ds(0, 16)]
            return c
        lax.fori_loop(0, _CH, merge, 0)
        pltpu.sync_copy(erows, acc.at[didx], add=True)
        return carry
    lax.fori_loop(0, nblk, chunk, 0)

    plsc.subcore_barrier()
    pltpu.sync_copy(acc.at[pl.ds(row0, _NPW)],
                    out_hbm.at[cid, pl.ds(row0, _NPW)])


@functools.cache
def _get_sc_seg_gather():
    return functools.partial(
        pl.kernel,
        out_type=jax.ShapeDtypeStruct((2, _N, _W), jnp.float32),
        mesh=plsc.VectorSubcoreMesh(core_axis_name="c", subcore_axis_name="s"),
        scratch_types=[
            pltpu.VMEM((_CH,), jnp.int32),
            pltpu.VMEM((_CH,), jnp.int32),
            pltpu.VMEM((_CH, _W), jnp.float32),
            pltpu.VMEM_SHARED((_N, _W), jnp.float32),
            pltpu.SemaphoreType.DMA,
        ],
    )(_sc_seg_gather_body)


def _sc_seg_gather_body(x_hbm, src_hbm, dst_hbm, out_hbm,
                        sidx, didx, rows, acc, sem):
    cid = lax.axis_index("c")
    sid = lax.axis_index("s")
    wid = sid * 2 + cid
    row0 = pl.multiple_of(jnp.minimum(sid * _NPW, _N - _NPW), 8)
    _zero_buf(rows)
    _zero_shared_rows(rows, acc, row0)
    plsc.subcore_barrier()

    nblk, start = _tile_chunks(wid)

    def chunk(k, carry):
        base = pl.multiple_of(start + k * _CH, 8)
        pltpu.sync_copy(src_hbm.at[pl.ds(base, _CH)], sidx)
        pltpu.sync_copy(dst_hbm.at[pl.ds(base, _CH)], didx)
        pltpu.async_copy(x_hbm.at[sidx], rows, sem).wait()
        pltpu.sync_copy(rows, acc.at[didx], add=True)
        return carry
    lax.fori_loop(0, nblk, chunk, 0)

    plsc.subcore_barrier()
    pltpu.sync_copy(acc.at[pl.ds(row0, _NPW)],
                    out_hbm.at[cid, pl.ds(row0, _NPW)])


def kernel(node_feature, edge_feature, edge_index,
           Wn, bn, We1, be1, We2, be2, We3, be3,
           netW1, netb1, linW1, linb1,
           netW2, netb2, linW2, linb2,
           netW3, netb3, linW3, linb3,
           netW4, netb4, linW4, linb4,
           netW5, netb5, linW5, linb5,
           Wlast, blast):
    src = edge_index[0]
    dst = edge_index[1]

    e_aug = pl.pallas_call(
        _edge_mlp_block,
        grid=(_E // _BE,),
        in_specs=[_rows((_BE, 301)),
                  _full((301, 128)), _full((1, 128)),
                  _full((128, 64)), _full((1, 64)),
                  _full((64, 32)), _full((1, 32))],
        out_specs=_rows((_BE, _W)),
        out_shape=jax.ShapeDtypeStruct((_E, _W), jnp.float32),
    )(edge_feature, We1.T, be1.reshape(1, -1),
      We2.T, be2.reshape(1, -1), We3.T, be3.reshape(1, -1))

    h = pl.pallas_call(
        _node_block,
        grid=(1,),
        in_specs=[_full((_N, 3)), _full((3, 16)), _full((1, 16))],
        out_specs=_full((_N, _W)),
        out_shape=jax.ShapeDtypeStruct((_N, _W), jnp.float32),
    )(node_feature, Wn.T, bn.reshape(1, -1))

    ep = _get_sc_layer1()(e_aug, h, src, dst)

    convs = [(netW1, netb1, linW1, linb1), (netW2, netb2, linW2, linb2),
             (netW3, netb3, linW3, linb3), (netW4, netb4, linW4, linb4),
             (netW5, netb5, linW5, linb5)]

    x = h
    gp = ep
    out2d = None
    for li, (nW, nb, lW, lb) in enumerate(convs):
        f_in = lW.shape[1]
        wx_t = nW[:, :f_in].T
        we_t = nW[:, f_in:].T
        we = jnp.zeros((_W, 32), jnp.float32)
        we = we.at[0:32].set(we_t).at[32].set(nb)
        wx = jnp.zeros((_W, 32), jnp.float32)
        if li == 0:
            we = we.at[_HC:_HC + f_in].set(wx_t)
        else:
            wx = wx.at[0:f_in].set(wx_t)
        wl = jnp.zeros((_W, 32), jnp.float32).at[0:f_in].set(lW.T)
        lbr = lb.reshape(1, -1)

        if li > 0:
            gp = _get_sc_seg_gather()(x, src, dst)

        common = [_rows((_BN, _W)),
                  _part(0, (_BN, _W)), _part(1, (_BN, _W)),
                  _part(0, (_BN, _W)), _part(1, (_BN, _W)),
                  _full((_W, 32)), _full((_W, 32)), _full((_W, 32)),
                  _full((1, 32))]
        if li < 4:
            x = pl.pallas_call(
                _layer_block,
                grid=(_N // _BN,),
                in_specs=common,
                out_specs=_rows((_BN, _W)),
                out_shape=jax.ShapeDtypeStruct((_N, _W), jnp.float32),
            )(x, ep, ep, gp, gp, we, wx, wl, lbr)
        else:
            out2d = pl.pallas_call(
                _final_block,
                grid=(_N // _BN,),
                in_specs=common + [_full((1, 32)), _full((1, 1))],
                out_specs=_rows((_BN, 1)),
                out_shape=jax.ShapeDtypeStruct((_N, 1), jnp.float32),
            )(x, ep, ep, gp, gp, we, wx, wl, lbr, Wlast, blast.reshape(1, 1))

    return out2d[:, 0]


# speedup vs baseline: 3.8126x; 1.1855x over previous
import functools

import jax
import jax.numpy as jnp
from jax import lax
from jax.experimental import pallas as pl
from jax.experimental.pallas import tpu as pltpu
from jax.experimental.pallas import tpu_sc as plsc

_N = 10000
_E = 160000
_W = 128
_CH = 128
_NBLK = 39
_MAIN = _NBLK * _CH * 32
_NPW = 632
_BE = 2000
_BN = 2000


def _edge_mlp_block(ef, w1, b1, w2, b2, w3, b3, out):
    t = jnp.maximum(ef[...] @ w1[...] + b1[...], 0.0)
    t = jnp.maximum(t @ w2[...] + b2[...], 0.0)
    t = jnp.maximum(t @ w3[...] + b3[...], 0.0)
    n = t.shape[0]
    out[...] = jnp.concatenate(
        [t, jnp.ones((n, 1), jnp.float32), jnp.zeros((n, _W - 33), jnp.float32)],
        axis=1)


def _node_block(nf, wn, bn_, out):
    h = jnp.maximum(jnp.dot(nf[...], wn[...]) + bn_[...], 0.0)
    out[...] = jnp.concatenate(
        [h, jnp.zeros((h.shape[0], _W - 16), jnp.float32)], axis=1)


def _layer_block(x, e0, e1, g0, g1, we, wx, wl, lb, out):
    es = e0[0] + e1[0]
    g = g0[0] + g1[0]
    acc = jnp.dot(es, we[...]) + jnp.dot(g, wx[...]) \
        + jnp.dot(x[...], wl[...]) + lb[...]
    r = jnp.maximum(acc, 0.0)
    out[...] = jnp.concatenate(
        [r, jnp.zeros((r.shape[0], _W - 32), jnp.float32)], axis=1)


def _final_block(x, e0, e1, g0, g1, we, wx, wl, lb, wlast, blast, out):
    es = e0[0] + e1[0]
    g = g0[0] + g1[0]
    acc = jnp.dot(es, we[...]) + jnp.dot(g, wx[...]) \
        + jnp.dot(x[...], wl[...]) + lb[...]
    x5 = jnp.maximum(acc, 0.0)
    out[...] = jnp.sum(x5 * wlast[...], axis=1, keepdims=True) + blast[...]


def _full(shape):
    return pl.BlockSpec(shape, lambda i: (0,) * len(shape))


def _rows(shape):
    return pl.BlockSpec(shape, lambda i: (i,) + (0,) * (len(shape) - 1))


def _part(c, shape):
    return pl.BlockSpec((1,) + shape, lambda i, _c=c: (_c, i, 0))


def _zero_buf(buf):
    def body(i, carry):
        for j in range(_W // 16):
            buf[i, pl.ds(j * 16, 16)] = jnp.zeros((16,), jnp.float32)
        return carry
    lax.fori_loop(0, buf.shape[0], body, 0)


def _zero_shared_rows(buf, acc, row0):
    off = 0
    while off < _NPW:
        nr = min(_CH, _NPW - off)
        pltpu.sync_copy(buf.at[pl.ds(0, nr)], acc.at[pl.ds(row0 + off, nr)])
        off += nr


def _sc_prelude(buf, acc):
    cid = lax.axis_index("c")
    sid = lax.axis_index("s")
    wid = sid * 2 + cid
    row0 = pl.multiple_of(jnp.minimum(sid * _NPW, _N - _NPW), 8)
    _zero_buf(buf)
    _zero_shared_rows(buf, acc, row0)
    plsc.subcore_barrier()
    start = pl.multiple_of(wid * (_NBLK * _CH), 8)
    return cid, wid, row0, start


def _sc_copy_out(acc, out_hbm, cid, row0):
    plsc.subcore_barrier()
    pltpu.sync_copy(acc.at[pl.ds(row0, _NPW)],
                    out_hbm.at[cid, pl.ds(row0, _NPW)])


@functools.cache
def _get_sc_escatter():
    return functools.partial(
        pl.kernel,
        out_type=jax.ShapeDtypeStruct((2, _N, _W), jnp.float32),
        mesh=plsc.VectorSubcoreMesh(core_axis_name="c", subcore_axis_name="s"),
        scratch_types=[
            pltpu.VMEM((_CH,), jnp.int32),
            pltpu.VMEM((_CH,), jnp.int32),
            pltpu.VMEM((_CH, _W), jnp.float32),
            pltpu.VMEM((_CH, _W), jnp.float32),
            pltpu.VMEM_SHARED((_N, _W), jnp.float32),
            pltpu.SemaphoreType.DMA,
            pltpu.SemaphoreType.DMA,
        ],
    )(_sc_escatter_body)


def _sc_escatter_body(e_hbm, dst_hbm, out_hbm,
                      didx0, didx1, erows0, erows1, acc, semi, semc):
    cid, wid, row0, start = _sc_prelude(erows0, acc)

    def pair(j, carry):
        b0 = pl.multiple_of(start + (2 * j) * _CH, 8)
        b1 = pl.multiple_of(b0 + _CH, 8)
        l0 = pltpu.async_copy(dst_hbm.at[pl.ds(b0, _CH)], didx0, semi)
        l1 = pltpu.async_copy(e_hbm.at[pl.ds(b0, _CH)], erows0, semi)
        l2 = pltpu.async_copy(dst_hbm.at[pl.ds(b1, _CH)], didx1, semi)
        l3 = pltpu.async_copy(e_hbm.at[pl.ds(b1, _CH)], erows1, semi)
        l0.wait(); l1.wait(); l2.wait(); l3.wait()
        s0 = pltpu.async_copy(erows0, acc.at[didx0], semc, add=True)
        s1 = pltpu.async_copy(erows1, acc.at[didx1], semc, add=True)
        s0.wait(); s1.wait()
        return carry
    lax.fori_loop(0, _NBLK // 2, pair, 0)

    bt = pl.multiple_of(start + (_NBLK - 1) * _CH, 8)
    pltpu.sync_copy(dst_hbm.at[pl.ds(bt, _CH)], didx0)
    pltpu.sync_copy(e_hbm.at[pl.ds(bt, _CH)], erows0)
    pltpu.sync_copy(erows0, acc.at[didx0], add=True)

    @pl.when(wid < 2)
    def _():
        be = pl.multiple_of(_MAIN + wid * _CH, 8)
        pltpu.sync_copy(dst_hbm.at[pl.ds(be, _CH)], didx1)
        pltpu.sync_copy(e_hbm.at[pl.ds(be, _CH)], erows1)
        pltpu.sync_copy(erows1, acc.at[didx1], add=True)

    _sc_copy_out(acc, out_hbm, cid, row0)


@functools.cache
def _get_sc_seg_gather():
    return functools.partial(
        pl.kernel,
        out_type=jax.ShapeDtypeStruct((2, _N, _W), jnp.float32),
        mesh=plsc.VectorSubcoreMesh(core_axis_name="c", subcore_axis_name="s"),
        scratch_types=[
            pltpu.VMEM((_CH,), jnp.int32),
            pltpu.VMEM((_CH,), jnp.int32),
            pltpu.VMEM((_CH,), jnp.int32),
            pltpu.VMEM((_CH,), jnp.int32),
            pltpu.VMEM((_CH, _W), jnp.float32),
            pltpu.VMEM((_CH, _W), jnp.float32),
            pltpu.VMEM_SHARED((_N, _W), jnp.float32),
            pltpu.SemaphoreType.DMA,
            pltpu.SemaphoreType.DMA,
            pltpu.SemaphoreType.DMA,
            pltpu.SemaphoreType.DMA,
        ],
    )(_sc_seg_gather_body)


def _sc_seg_gather_body(x_hbm, src_hbm, dst_hbm, out_hbm,
                        sidx0, didx0, sidx1, didx1, rows0, rows1, acc,
                        semi, semg0, semg1, semc):
    cid, wid, row0, start = _sc_prelude(rows0, acc)

    def pair(j, carry):
        b0 = pl.multiple_of(start + (2 * j) * _CH, 8)
        b1 = pl.multiple_of(b0 + _CH, 8)
        l0 = pltpu.async_copy(src_hbm.at[pl.ds(b0, _CH)], sidx0, semi)
        l1 = pltpu.async_copy(dst_hbm.at[pl.ds(b0, _CH)], didx0, semi)
        l2 = pltpu.async_copy(src_hbm.at[pl.ds(b1, _CH)], sidx1, semi)
        l3 = pltpu.async_copy(dst_hbm.at[pl.ds(b1, _CH)], didx1, semi)
        l0.wait(); l1.wait(); l2.wait(); l3.wait()
        g0 = pltpu.async_copy(x_hbm.at[sidx0], rows0, semg0)
        g1 = pltpu.async_copy(x_hbm.at[sidx1], rows1, semg1)
        g0.wait()
        s0 = pltpu.async_copy(rows0, acc.at[didx0], semc, add=True)
        g1.wait()
        s1 = pltpu.async_copy(rows1, acc.at[didx1], semc, add=True)
        s0.wait(); s1.wait()
        return carry
    lax.fori_loop(0, _NBLK // 2, pair, 0)

    bt = pl.multiple_of(start + (_NBLK - 1) * _CH, 8)
    pltpu.sync_copy(src_hbm.at[pl.ds(bt, _CH)], sidx0)
    pltpu.sync_copy(dst_hbm.at[pl.ds(bt, _CH)], didx0)
    pltpu.async_copy(x_hbm.at[sidx0], rows0, semg0).wait()
    pltpu.sync_copy(rows0, acc.at[didx0], add=True)

    @pl.when(wid < 2)
    def _():
        be = pl.multiple_of(_MAIN + wid * _CH, 8)
        pltpu.sync_copy(src_hbm.at[pl.ds(be, _CH)], sidx1)
        pltpu.sync_copy(dst_hbm.at[pl.ds(be, _CH)], didx1)
        pltpu.async_copy(x_hbm.at[sidx1], rows1, semg1).wait()
        pltpu.sync_copy(rows1, acc.at[didx1], add=True)

    _sc_copy_out(acc, out_hbm, cid, row0)


def kernel(node_feature, edge_feature, edge_index,
           Wn, bn, We1, be1, We2, be2, We3, be3,
           netW1, netb1, linW1, linb1,
           netW2, netb2, linW2, linb2,
           netW3, netb3, linW3, linb3,
           netW4, netb4, linW4, linb4,
           netW5, netb5, linW5, linb5,
           Wlast, blast):
    src = edge_index[0]
    dst = edge_index[1]

    e_aug = pl.pallas_call(
        _edge_mlp_block,
        grid=(_E // _BE,),
        in_specs=[_rows((_BE, 301)),
                  _full((301, 128)), _full((1, 128)),
                  _full((128, 64)), _full((1, 64)),
                  _full((64, 32)), _full((1, 32))],
        out_specs=_rows((_BE, _W)),
        out_shape=jax.ShapeDtypeStruct((_E, _W), jnp.float32),
    )(edge_feature, We1.T, be1.reshape(1, -1),
      We2.T, be2.reshape(1, -1), We3.T, be3.reshape(1, -1))

    h = pl.pallas_call(
        _node_block,
        grid=(1,),
        in_specs=[_full((_N, 3)), _full((3, 16)), _full((1, 16))],
        out_specs=_full((_N, _W)),
        out_shape=jax.ShapeDtypeStruct((_N, _W), jnp.float32),
    )(node_feature, Wn.T, bn.reshape(1, -1))

    ep = _get_sc_escatter()(e_aug, dst)

    convs = [(netW1, netb1, linW1, linb1), (netW2, netb2, linW2, linb2),
             (netW3, netb3, linW3, linb3), (netW4, netb4, linW4, linb4),
             (netW5, netb5, linW5, linb5)]

    x = h
    out2d = None
    for li, (nW, nb, lW, lb) in enumerate(convs):
        f_in = lW.shape[1]
        we = jnp.zeros((_W, 32), jnp.float32)
        we = we.at[0:32].set(nW[:, f_in:].T).at[32].set(nb)
        wx = jnp.zeros((_W, 32), jnp.float32).at[0:f_in].set(nW[:, :f_in].T)
        wl = jnp.zeros((_W, 32), jnp.float32).at[0:f_in].set(lW.T)
        lbr = lb.reshape(1, -1)

        gp = _get_sc_seg_gather()(x, src, dst)

        common = [_rows((_BN, _W)),
                  _part(0, (_BN, _W)), _part(1, (_BN, _W)),
                  _part(0, (_BN, _W)), _part(1, (_BN, _W)),
                  _full((_W, 32)), _full((_W, 32)), _full((_W, 32)),
                  _full((1, 32))]
        if li < 4:
            x = pl.pallas_call(
                _layer_block,
                grid=(_N // _BN,),
                in_specs=common,
                out_specs=_rows((_BN, _W)),
                out_shape=jax.ShapeDtypeStruct((_N, _W), jnp.float32),
            )(x, ep, ep, gp, gp, we, wx, wl, lbr)
        else:
            out2d = pl.pallas_call(
                _final_block,
                grid=(_N // _BN,),
                in_specs=common + [_full((1, 32)), _full((1, 1))],
                out_specs=_rows((_BN, 1)),
                out_shape=jax.ShapeDtypeStruct((_N, 1), jnp.float32),
            )(x, ep, ep, gp, gp, we, wx, wl, lbr, Wlast, blast.reshape(1, 1))

    return out2d[:, 0]


# speedup vs baseline: 3.8381x; 1.0067x over previous
import functools

import jax
import jax.numpy as jnp
from jax import lax
from jax.experimental import pallas as pl
from jax.experimental.pallas import tpu as pltpu
from jax.experimental.pallas import tpu_sc as plsc

_N = 10000
_E = 160000
_W = 128
_CH = 128
_NBLK = 39
_MAIN = _NBLK * _CH * 32
_NPW = 632
_BE = 2000
_BN = 2000


def _edge_mlp_block(ef, w1, b1, w2, b2, w3, b3, out):
    t = jnp.maximum(ef[...] @ w1[...] + b1[...], 0.0)
    t = jnp.maximum(t @ w2[...] + b2[...], 0.0)
    t = jnp.maximum(t @ w3[...] + b3[...], 0.0)
    n = t.shape[0]
    out[...] = jnp.concatenate(
        [t, jnp.ones((n, 1), jnp.float32), jnp.zeros((n, _W - 33), jnp.float32)],
        axis=1)


def _node_block(nf, wn, bn_, out):
    h = jnp.maximum(jnp.dot(nf[...], wn[...]) + bn_[...], 0.0)
    out[...] = jnp.concatenate(
        [h, jnp.zeros((h.shape[0], _W - 16), jnp.float32)], axis=1)


def _layer_block(x, e0, e1, g0, g1, we, wx, wl, lb, out):
    es = e0[0] + e1[0]
    g = g0[0] + g1[0]
    acc = jnp.dot(es, we[...]) + jnp.dot(g, wx[...]) \
        + jnp.dot(x[...], wl[...]) + lb[...]
    r = jnp.maximum(acc, 0.0)
    out[...] = jnp.concatenate(
        [r, jnp.zeros((r.shape[0], _W - 32), jnp.float32)], axis=1)


def _final_block(x, e0, e1, g0, g1, we, wx, wl, lb, wlast, blast, out):
    es = e0[0] + e1[0]
    g = g0[0] + g1[0]
    acc = jnp.dot(es, we[...]) + jnp.dot(g, wx[...]) \
        + jnp.dot(x[...], wl[...]) + lb[...]
    x5 = jnp.maximum(acc, 0.0)
    out[...] = jnp.sum(x5 * wlast[...], axis=1, keepdims=True) + blast[...]


def _full(shape):
    return pl.BlockSpec(shape, lambda i: (0,) * len(shape))


def _rows(shape):
    return pl.BlockSpec(shape, lambda i: (i,) + (0,) * (len(shape) - 1))


def _part(c, shape):
    return pl.BlockSpec((1,) + shape, lambda i, _c=c: (_c, i, 0))


def _zero_buf(buf):
    def body(i, carry):
        for j in range(_W // 16):
            buf[i, pl.ds(j * 16, 16)] = jnp.zeros((16,), jnp.float32)
        return carry
    lax.fori_loop(0, buf.shape[0], body, 0)


def _zero_shared_rows(buf, acc, row0):
    off = 0
    while off < _NPW:
        nr = min(_CH, _NPW - off)
        pltpu.sync_copy(buf.at[pl.ds(0, nr)], acc.at[pl.ds(row0 + off, nr)])
        off += nr


def _sc_prelude(buf, acc):
    cid = lax.axis_index("c")
    sid = lax.axis_index("s")
    wid = sid * 2 + cid
    row0 = pl.multiple_of(jnp.minimum(sid * _NPW, _N - _NPW), 8)
    _zero_buf(buf)
    _zero_shared_rows(buf, acc, row0)
    plsc.subcore_barrier()
    start = pl.multiple_of(wid * (_NBLK * _CH), 8)
    return cid, wid, row0, start


def _sc_copy_out(acc, out_hbm, cid, row0):
    plsc.subcore_barrier()
    pltpu.sync_copy(acc.at[pl.ds(row0, _NPW)],
                    out_hbm.at[cid, pl.ds(row0, _NPW)])


@functools.cache
def _get_sc_escatter():
    return functools.partial(
        pl.kernel,
        out_type=jax.ShapeDtypeStruct((2, _N, _W), jnp.float32),
        mesh=plsc.VectorSubcoreMesh(core_axis_name="c", subcore_axis_name="s"),
        scratch_types=[
            pltpu.VMEM((_CH,), jnp.int32),
            pltpu.VMEM((_CH,), jnp.int32),
            pltpu.VMEM((_CH,), jnp.int32),
            pltpu.VMEM((_CH, _W), jnp.float32),
            pltpu.VMEM((_CH, _W), jnp.float32),
            pltpu.VMEM((_CH, _W), jnp.float32),
            pltpu.VMEM_SHARED((_N, _W), jnp.float32),
            pltpu.SemaphoreType.DMA,
            pltpu.SemaphoreType.DMA,
        ],
    )(_sc_escatter_body)


def _sc_escatter_body(e_hbm, dst_hbm, out_hbm,
                      didx0, didx1, didx2, erows0, erows1, erows2, acc,
                      semi, semc):
    cid, wid, row0, start = _sc_prelude(erows0, acc)

    def triple(j, carry):
        b0 = pl.multiple_of(start + (3 * j) * _CH, 8)
        b1 = pl.multiple_of(b0 + _CH, 8)
        b2 = pl.multiple_of(b0 + 2 * _CH, 8)
        ls = [pltpu.async_copy(dst_hbm.at[pl.ds(b0, _CH)], didx0, semi),
              pltpu.async_copy(e_hbm.at[pl.ds(b0, _CH)], erows0, semi),
              pltpu.async_copy(dst_hbm.at[pl.ds(b1, _CH)], didx1, semi),
              pltpu.async_copy(e_hbm.at[pl.ds(b1, _CH)], erows1, semi),
              pltpu.async_copy(dst_hbm.at[pl.ds(b2, _CH)], didx2, semi),
              pltpu.async_copy(e_hbm.at[pl.ds(b2, _CH)], erows2, semi)]
        for l in ls:
            l.wait()
        s0 = pltpu.async_copy(erows0, acc.at[didx0], semc, add=True)
        s1 = pltpu.async_copy(erows1, acc.at[didx1], semc, add=True)
        s2 = pltpu.async_copy(erows2, acc.at[didx2], semc, add=True)
        s0.wait(); s1.wait(); s2.wait()
        return carry
    lax.fori_loop(0, _NBLK // 3, triple, 0)

    @pl.when(wid < 2)
    def _():
        be = pl.multiple_of(_MAIN + wid * _CH, 8)
        pltpu.sync_copy(dst_hbm.at[pl.ds(be, _CH)], didx1)
        pltpu.sync_copy(e_hbm.at[pl.ds(be, _CH)], erows1)
        pltpu.sync_copy(erows1, acc.at[didx1], add=True)

    _sc_copy_out(acc, out_hbm, cid, row0)


@functools.cache
def _get_sc_seg_gather():
    return functools.partial(
        pl.kernel,
        out_type=jax.ShapeDtypeStruct((2, _N, _W), jnp.float32),
        mesh=plsc.VectorSubcoreMesh(core_axis_name="c", subcore_axis_name="s"),
        scratch_types=[
            pltpu.VMEM((_CH,), jnp.int32),
            pltpu.VMEM((_CH,), jnp.int32),
            pltpu.VMEM((_CH,), jnp.int32),
            pltpu.VMEM((_CH,), jnp.int32),
            pltpu.VMEM((_CH,), jnp.int32),
            pltpu.VMEM((_CH,), jnp.int32),
            pltpu.VMEM((_CH, _W), jnp.float32),
            pltpu.VMEM((_CH, _W), jnp.float32),
            pltpu.VMEM((_CH, _W), jnp.float32),
            pltpu.VMEM_SHARED((_N, _W), jnp.float32),
            pltpu.SemaphoreType.DMA,
            pltpu.SemaphoreType.DMA,
            pltpu.SemaphoreType.DMA,
            pltpu.SemaphoreType.DMA,
            pltpu.SemaphoreType.DMA,
        ],
    )(_sc_seg_gather_body)


def _sc_seg_gather_body(x_hbm, src_hbm, dst_hbm, out_hbm,
                        sidx0, didx0, sidx1, didx1, sidx2, didx2,
                        rows0, rows1, rows2, acc,
                        semi, semg0, semg1, semg2, semc):
    cid, wid, row0, start = _sc_prelude(rows0, acc)

    def pair(j, carry):
        b0 = pl.multiple_of(start + (2 * j) * _CH, 8)
        b1 = pl.multiple_of(b0 + _CH, 8)
        l0 = pltpu.async_copy(src_hbm.at[pl.ds(b0, _CH)], sidx0, semi)
        l1 = pltpu.async_copy(dst_hbm.at[pl.ds(b0, _CH)], didx0, semi)
        l2 = pltpu.async_copy(src_hbm.at[pl.ds(b1, _CH)], sidx1, semi)
        l3 = pltpu.async_copy(dst_hbm.at[pl.ds(b1, _CH)], didx1, semi)
        l0.wait(); l1.wait(); l2.wait(); l3.wait()
        g0 = pltpu.async_copy(x_hbm.at[sidx0], rows0, semg0)
        g1 = pltpu.async_copy(x_hbm.at[sidx1], rows1, semg1)
        g0.wait()
        s0 = pltpu.async_copy(rows0, acc.at[didx0], semc, add=True)
        g1.wait()
        s1 = pltpu.async_copy(rows1, acc.at[didx1], semc, add=True)
        s0.wait(); s1.wait()
        return carry
    lax.fori_loop(0, _NBLK // 2, pair, 0)

    bt = pl.multiple_of(start + (_NBLK - 1) * _CH, 8)
    pltpu.sync_copy(src_hbm.at[pl.ds(bt, _CH)], sidx0)
    pltpu.sync_copy(dst_hbm.at[pl.ds(bt, _CH)], didx0)
    pltpu.async_copy(x_hbm.at[sidx0], rows0, semg0).wait()
    pltpu.sync_copy(rows0, acc.at[didx0], add=True)

    @pl.when(wid < 2)
    def _():
        be = pl.multiple_of(_MAIN + wid * _CH, 8)
        pltpu.sync_copy(src_hbm.at[pl.ds(be, _CH)], sidx1)
        pltpu.sync_copy(dst_hbm.at[pl.ds(be, _CH)], didx1)
        pltpu.async_copy(x_hbm.at[sidx1], rows1, semg1).wait()
        pltpu.sync_copy(rows1, acc.at[didx1], add=True)

    _sc_copy_out(acc, out_hbm, cid, row0)


def kernel(node_feature, edge_feature, edge_index,
           Wn, bn, We1, be1, We2, be2, We3, be3,
           netW1, netb1, linW1, linb1,
           netW2, netb2, linW2, linb2,
           netW3, netb3, linW3, linb3,
           netW4, netb4, linW4, linb4,
           netW5, netb5, linW5, linb5,
           Wlast, blast):
    src = edge_index[0]
    dst = edge_index[1]

    h = pl.pallas_call(
        _node_block,
        grid=(1,),
        in_specs=[_full((_N, 3)), _full((3, 16)), _full((1, 16))],
        out_specs=_full((_N, _W)),
        out_shape=jax.ShapeDtypeStruct((_N, _W), jnp.float32),
    )(node_feature, Wn.T, bn.reshape(1, -1))

    gp1 = _get_sc_seg_gather()(h, src, dst)

    e_aug = pl.pallas_call(
        _edge_mlp_block,
        grid=(_E // _BE,),
        in_specs=[_rows((_BE, 301)),
                  _full((301, 128)), _full((1, 128)),
                  _full((128, 64)), _full((1, 64)),
                  _full((64, 32)), _full((1, 32))],
        out_specs=_rows((_BE, _W)),
        out_shape=jax.ShapeDtypeStruct((_E, _W), jnp.float32),
    )(edge_feature, We1.T, be1.reshape(1, -1),
      We2.T, be2.reshape(1, -1), We3.T, be3.reshape(1, -1))

    ep = _get_sc_escatter()(e_aug, dst)

    convs = [(netW1, netb1, linW1, linb1), (netW2, netb2, linW2, linb2),
             (netW3, netb3, linW3, linb3), (netW4, netb4, linW4, linb4),
             (netW5, netb5, linW5, linb5)]

    x = h
    out2d = None
    for li, (nW, nb, lW, lb) in enumerate(convs):
        f_in = lW.shape[1]
        we = jnp.zeros((_W, 32), jnp.float32)
        we = we.at[0:32].set(nW[:, f_in:].T).at[32].set(nb)
        wx = jnp.zeros((_W, 32), jnp.float32).at[0:f_in].set(nW[:, :f_in].T)
        wl = jnp.zeros((_W, 32), jnp.float32).at[0:f_in].set(lW.T)
        lbr = lb.reshape(1, -1)

        gp = gp1 if li == 0 else _get_sc_seg_gather()(x, src, dst)

        common = [_rows((_BN, _W)),
                  _part(0, (_BN, _W)), _part(1, (_BN, _W)),
                  _part(0, (_BN, _W)), _part(1, (_BN, _W)),
                  _full((_W, 32)), _full((_W, 32)), _full((_W, 32)),
                  _full((1, 32))]
        if li < 4:
            x = pl.pallas_call(
                _layer_block,
                grid=(_N // _BN,),
                in_specs=common,
                out_specs=_rows((_BN, _W)),
                out_shape=jax.ShapeDtypeStruct((_N, _W), jnp.float32),
            )(x, ep, ep, gp, gp, we, wx, wl, lbr)
        else:
            out2d = pl.pallas_call(
                _final_block,
                grid=(_N // _BN,),
                in_specs=common + [_full((1, 32)), _full((1, 1))],
                out_specs=_rows((_BN, 1)),
                out_shape=jax.ShapeDtypeStruct((_N, 1), jnp.float32),
            )(x, ep, ep, gp, gp, we, wx, wl, lbr, Wlast, blast.reshape(1, 1))

    return out2d[:, 0]
